# baseline (speedup 1.0000x reference)
import functools

import jax
import jax.numpy as jnp
from jax import lax
from jax.experimental import pallas as pl
from jax.experimental.pallas import tpu as pltpu
from jax.experimental.pallas import tpu_sc as plsc

BATCH = 4096
HIST = 200
EMBED = 64
PAIRS = 128
SPLIT = 104

NBUF = 2
UNROLL = 8


def _make_pooling_kernel():
  info = plsc.get_sparse_core_info()
  nw = info.num_cores * info.num_subcores
  b_per_w = BATCH // nw

  mesh = plsc.VectorSubcoreMesh(core_axis_name="c", subcore_axis_name="s")

  @functools.partial(
      pl.kernel,
      mesh=mesh,
      out_type=jax.ShapeDtypeStruct((BATCH * EMBED,), jnp.float32),
      scratch_types=[
          pltpu.VMEM((b_per_w, HIST), jnp.int32),
          pltpu.VMEM((b_per_w * HIST,), jnp.int32),
          pltpu.VMEM((NBUF, HIST, PAIRS), jnp.float32),
          pltpu.VMEM((b_per_w * EMBED,), jnp.float32),
          [pltpu.SemaphoreType.DMA] * NBUF,
      ],
  )
  def pool(x_hbm, table_hbm, out_hbm, idx_v, rp_v, rows_v, pooled_v, sems):
    wid = lax.axis_index("s") * info.num_cores + lax.axis_index("c")
    base = wid * b_per_w

    pltpu.sync_copy(x_hbm.at[pl.ds(base, b_per_w)], idx_v)

    col_offs = tuple(range(0, HIST - 15, 16)) + (HIST - 16,)

    def rp_body(b, _):
      for c in col_offs:
        rp_v[pl.ds(b * HIST + c, 16)] = lax.shift_right_logical(
            idx_v[b, pl.ds(c, 16)], 1)
      return ()

    lax.fori_loop(0, b_per_w, rp_body, ())

    def fire(b, p):
      pltpu.async_copy(
          table_hbm.at[rp_v.at[pl.ds(b * HIST, SPLIT)]],
          rows_v.at[p, pl.ds(0, SPLIT)], sems[p])
      pltpu.async_copy(
          table_hbm.at[rp_v.at[pl.ds(b * HIST + SPLIT, HIST - SPLIT)]],
          rows_v.at[p, pl.ds(SPLIT, HIST - SPLIT)], sems[p])

    def consume(b, p):
      pltpu.make_async_copy(
          table_hbm.at[pl.ds(0, HIST)], rows_v.at[p], sems[p]).wait()

      def add_rows(l0, lanes, acc):
        a0, a1, a2, a3 = acc
        offs = (idx_v[b, pl.ds(l0, 16)] & 1) << 6
        for u in lanes:
          off = offs[u]
          a0 = a0 + rows_v[p, l0 + u, pl.ds(off, 16)]
          a1 = a1 + rows_v[p, l0 + u, pl.ds(off + 16, 16)]
          a2 = a2 + rows_v[p, l0 + u, pl.ds(off + 32, 16)]
          a3 = a3 + rows_v[p, l0 + u, pl.ds(off + 48, 16)]
        return (a0, a1, a2, a3)

      def sum_body(i, acc):
        return add_rows(i * 16, range(16), acc)

      zero = jnp.zeros((16,), jnp.float32)
      acc = lax.fori_loop(
          0, HIST // 16, sum_body, (zero, zero, zero, zero))
      a0, a1, a2, a3 = add_rows(HIST - 16, range(8, 16), acc)
      pooled_v[pl.ds(b * EMBED, 16)] = a0
      pooled_v[pl.ds(b * EMBED + 16, 16)] = a1
      pooled_v[pl.ds(b * EMBED + 32, 16)] = a2
      pooled_v[pl.ds(b * EMBED + 48, 16)] = a3

    for p in range(NBUF):
      fire(p, p)

    def group_body(g, _):
      for p in range(NBUF):
        b = g * NBUF + p
        consume(b, p)
        fire(b + NBUF, p)
      return ()

    n_groups = b_per_w // NBUF
    lax.fori_loop(0, n_groups - 1, group_body, ())

    for p in range(NBUF):
      consume((n_groups - 1) * NBUF + p, p)

    pltpu.sync_copy(pooled_v, out_hbm.at[pl.ds(base * EMBED, b_per_w * EMBED)])

  return pool


_pooling_kernel = _make_pooling_kernel()


def _mlp_kernel(pooled_ref, w1_ref, b1_ref, w2_ref, b2_ref, out_ref):
  h = jnp.dot(pooled_ref[...], w1_ref[...],
              preferred_element_type=jnp.float32)
  h = jnp.maximum(h + b1_ref[...], 0.0)
  out_ref[...] = jnp.dot(h, w2_ref[...],
                         preferred_element_type=jnp.float32) + b2_ref[...]


@jax.jit
def kernel(x, table, W1, b1, W2, b2):
  table2 = table.reshape(-1, PAIRS)
  pooled = _pooling_kernel(x, table2).reshape(BATCH, EMBED)
  out = pl.pallas_call(
      _mlp_kernel,
      out_shape=jax.ShapeDtypeStruct((BATCH, 4), jnp.float32),
  )(pooled, W1, b1.reshape(1, 100), W2, b2.reshape(1, 4))
  return out


# speedup vs baseline: 1.1906x; 1.1906x over previous
import functools

import jax
import jax.numpy as jnp
from jax import lax
from jax.experimental import pallas as pl
from jax.experimental.pallas import tpu as pltpu
from jax.experimental.pallas import tpu_sc as plsc

BATCH = 4096
HIST = 200
EMBED = 64
VOCAB = 1000000
ROWPAD = 128
SPLIT = 104

NBUF = 2
RB = 2048


def _repack_kernel(tt_ref, out_ref):
  t = tt_ref[...]
  out_ref[...] = jnp.concatenate(
      [t.T, jnp.zeros((RB, ROWPAD - EMBED), jnp.float32)], axis=1)


def _repack(table_t):
  grid = (VOCAB + RB - 1) // RB
  return pl.pallas_call(
      _repack_kernel,
      grid=(grid,),
      in_specs=[pl.BlockSpec((EMBED, RB), lambda c: (0, c))],
      out_specs=pl.BlockSpec((RB, ROWPAD), lambda c: (c, 0)),
      out_shape=jax.ShapeDtypeStruct((VOCAB, ROWPAD), jnp.float32),
  )(table_t)


def _make_pooling_kernel():
  info = plsc.get_sparse_core_info()
  nw = info.num_cores * info.num_subcores
  b_per_w = BATCH // nw

  mesh = plsc.VectorSubcoreMesh(core_axis_name="c", subcore_axis_name="s")

  @functools.partial(
      pl.kernel,
      mesh=mesh,
      out_type=jax.ShapeDtypeStruct((BATCH * EMBED,), jnp.float32),
      scratch_types=[
          pltpu.VMEM((b_per_w * HIST,), jnp.int32),
          pltpu.VMEM((NBUF, HIST, ROWPAD), jnp.float32),
          pltpu.VMEM((b_per_w * EMBED,), jnp.float32),
          [pltpu.SemaphoreType.DMA] * NBUF,
      ],
  )
  def pool(x_hbm, table_hbm, out_hbm, idx_v, rows_v, pooled_v, sems):
    wid = lax.axis_index("s") * info.num_cores + lax.axis_index("c")
    base = wid * b_per_w

    pltpu.sync_copy(x_hbm.at[pl.ds(base * HIST, b_per_w * HIST)], idx_v)

    def fire(b, p):
      pltpu.async_copy(
          table_hbm.at[idx_v.at[pl.ds(b * HIST, SPLIT)]],
          rows_v.at[p, pl.ds(0, SPLIT)], sems[p])
      pltpu.async_copy(
          table_hbm.at[idx_v.at[pl.ds(b * HIST + SPLIT, HIST - SPLIT)]],
          rows_v.at[p, pl.ds(SPLIT, HIST - SPLIT)], sems[p])

    def consume(b, p):
      pltpu.make_async_copy(
          table_hbm.at[pl.ds(0, HIST)], rows_v.at[p], sems[p]).wait()

      def sum_body(i, acc):
        a0, a1, a2, a3 = acc
        l0 = i * 8
        for u in range(8):
          a0 = a0 + rows_v[p, l0 + u, pl.ds(0, 16)]
          a1 = a1 + rows_v[p, l0 + u, pl.ds(16, 16)]
          a2 = a2 + rows_v[p, l0 + u, pl.ds(32, 16)]
          a3 = a3 + rows_v[p, l0 + u, pl.ds(48, 16)]
        return (a0, a1, a2, a3)

      zero = jnp.zeros((16,), jnp.float32)
      a0, a1, a2, a3 = lax.fori_loop(
          0, HIST // 8, sum_body, (zero, zero, zero, zero))
      pooled_v[pl.ds(b * EMBED, 16)] = a0
      pooled_v[pl.ds(b * EMBED + 16, 16)] = a1
      pooled_v[pl.ds(b * EMBED + 32, 16)] = a2
      pooled_v[pl.ds(b * EMBED + 48, 16)] = a3

    for p in range(NBUF):
      fire(p, p)

    def group_body(g, _):
      for p in range(NBUF):
        b = g * NBUF + p
        consume(b, p)
        fire(b + NBUF, p)
      return ()

    n_groups = b_per_w // NBUF
    lax.fori_loop(0, n_groups - 1, group_body, ())

    for p in range(NBUF):
      consume((n_groups - 1) * NBUF + p, p)

    pltpu.sync_copy(pooled_v, out_hbm.at[pl.ds(base * EMBED, b_per_w * EMBED)])

  return pool


_pooling_kernel = _make_pooling_kernel()


def _mlp_kernel(pooled_ref, w1_ref, b1_ref, w2_ref, b2_ref, out_ref):
  h = jnp.dot(pooled_ref[...], w1_ref[...],
              preferred_element_type=jnp.float32)
  h = jnp.maximum(h + b1_ref[...], 0.0)
  out_ref[...] = jnp.dot(h, w2_ref[...],
                         preferred_element_type=jnp.float32) + b2_ref[...]


@jax.jit
def kernel(x, table, W1, b1, W2, b2):
  table128 = _repack(table.T)
  pooled = _pooling_kernel(x.reshape(-1), table128).reshape(BATCH, EMBED)
  out = pl.pallas_call(
      _mlp_kernel,
      out_shape=jax.ShapeDtypeStruct((BATCH, 4), jnp.float32),
  )(pooled, W1, b1.reshape(1, 100), W2, b2.reshape(1, 4))
  return out
